# baseline (speedup 1.0000x reference)
import functools

import jax
import jax.numpy as jnp
from jax import lax
from jax.experimental import pallas as pl
from jax.experimental.pallas import tpu as pltpu
from jax.experimental.pallas import tpu_sc as plsc

B, N, D = 32, 576, 768
N_SLOTS = 8
LANES = 16
NV = N // LANES
NP = 640


def _gram_body(f_ref, g_ref, sal_ref):
    f = f_ref[0]
    norm = jnp.sqrt(jnp.sum(f * f, axis=1, keepdims=True))
    r = lax.dot_general(f, f, (((1,), (1,)), ((), ())),
                        preferred_element_type=jnp.float32)
    g_ref[0, :, :N] = r
    sal_ref[0, 0] = norm[:, 0]
    sal_ref[0, 1] = (1.0 / (norm + 1e-12))[:, 0]


def _tc_gram(features):
    return pl.pallas_call(
        _gram_body,
        grid=(B,),
        in_specs=[pl.BlockSpec((1, N, D), lambda b: (b, 0, 0))],
        out_specs=[
            pl.BlockSpec((1, N, NP), lambda b: (b, 0, 0)),
            pl.BlockSpec((1, 2, N), lambda b: (b, 0, 0)),
        ],
        out_shape=[
            jax.ShapeDtypeStruct((B, N, NP), jnp.float32),
            jax.ShapeDtypeStruct((B, 2, N), jnp.float32),
        ],
    )(features)


def _lane_gather(v, idx):
    return v.at[idx].get(mode="promise_in_bounds")


_UNROLL = 4
assert NV % _UNROLL == 0


def _merge(av, ai, aw, bv, bi, bw):
    better = (bv > av) | ((bv == av) & (bi < ai))
    return (jnp.where(better, bv, av), jnp.where(better, bi, ai),
            jnp.where(better, bw, aw))


def _sc_greedy(sal0_hbm, g_hbm, f_hbm, out_hbm, sal_v, inv_v, grow_v, idx_v,
               slots_v, sem):
    b = lax.axis_index("s") * 2 + lax.axis_index("c")
    pltpu.sync_copy(sal0_hbm.at[b, 0], sal_v)
    pltpu.sync_copy(sal0_hbm.at[b, 1], inv_v)
    iota = lax.iota(jnp.int32, LANES)
    neginf = jnp.float32(-jnp.inf)
    sel_vec = jnp.full((LANES,), b * N, jnp.int32)
    zero_i = jnp.zeros((LANES,), jnp.int32)
    ninf_v = jnp.full((LANES,), neginf)

    def argmax_lanes(carry_in, update_with_row, prev_idx, inv_i=None):
        def body(j, carry):
            accs = list(carry)
            for u in range(_UNROLL):
                jj = j * _UNROLL + u
                v = sal_v[pl.ds(jj * LANES, LANES)]
                w = inv_v[pl.ds(jj * LANES, LANES)]
                gi = jj * LANES + iota
                if update_with_row:
                    raw = grow_v[0, pl.ds(jj * LANES, LANES)]
                    sim = raw * w * inv_i
                    factor = 1.0 - jnp.clip(sim, 0.0, 1.0)
                    keep_inf = (gi == prev_idx) | (v == neginf)
                    v = jnp.where(keep_inf, neginf, v * factor)
                    sal_v[pl.ds(jj * LANES, LANES)] = v
                av, ai, aw = accs[3 * u], accs[3 * u + 1], accs[3 * u + 2]
                upd = v > av
                accs[3 * u] = jnp.where(upd, v, av)
                accs[3 * u + 1] = jnp.where(upd, gi, ai)
                accs[3 * u + 2] = jnp.where(upd, w, aw)
            return tuple(accs)

        carry = lax.fori_loop(0, NV // _UNROLL, body, carry_in)
        vmax, vidx, vinv = carry[0], carry[1], carry[2]
        for u in range(1, _UNROLL):
            vmax, vidx, vinv = _merge(vmax, vidx, vinv, carry[3 * u],
                                      carry[3 * u + 1], carry[3 * u + 2])
        for k in (1, 2, 4, 8):
            pv = _lane_gather(vmax, iota ^ k)
            pi = _lane_gather(vidx, iota ^ k)
            pw = _lane_gather(vinv, iota ^ k)
            vmax, vidx, vinv = _merge(vmax, vidx, vinv, pv, pi, pw)
        return vidx, vinv

    zero_f = jnp.zeros((LANES,), jnp.float32)
    init = tuple(x for _ in range(_UNROLL) for x in (ninf_v, zero_i, zero_f))
    idx_bcast, inv_bcast = argmax_lanes(init, False, None)
    for t in range(N_SLOTS):
        gidx_vec = idx_bcast + b * N
        sel_vec = jnp.where(iota == t, gidx_vec, sel_vec)
        if t == N_SLOTS - 1:
            break
        idx_v[...] = gidx_vec
        pltpu.async_copy(g_hbm.at[idx_v.at[pl.ds(0, 1)]], grow_v, sem).wait()
        idx_bcast, inv_bcast = argmax_lanes(init, True, idx_bcast, inv_bcast)

    idx_v[...] = sel_vec
    pltpu.async_copy(f_hbm.at[idx_v.at[pl.ds(0, N_SLOTS)]], slots_v,
                     sem).wait()
    pltpu.sync_copy(slots_v, out_hbm.at[b])


@functools.lru_cache(maxsize=1)
def _sc_greedy_kernel():
    mesh = plsc.VectorSubcoreMesh(core_axis_name="c", subcore_axis_name="s",
                                  num_cores=2, num_subcores=16)
    return pl.kernel(
        _sc_greedy,
        out_type=jax.ShapeDtypeStruct((B, N_SLOTS, D), jnp.float32),
        mesh=mesh,
        scratch_types=[
            pltpu.VMEM((N,), jnp.float32),
            pltpu.VMEM((N,), jnp.float32),
            pltpu.VMEM((1, NP), jnp.float32),
            pltpu.VMEM((LANES,), jnp.int32),
            pltpu.VMEM((N_SLOTS, D), jnp.float32),
            pltpu.SemaphoreType.DMA,
        ],
    )


@jax.jit
def kernel(features):
    g, sal0 = _tc_gram(features)
    g2 = g.reshape(B * N, NP)
    f2 = features.reshape(B * N, D)
    return _sc_greedy_kernel()(sal0, g2, f2)
